# baseline (speedup 1.0000x reference)
import functools

import jax
import jax.numpy as jnp
from jax import lax
from jax.experimental import pallas as pl
from jax.experimental.pallas import tpu as pltpu
from jax.experimental.pallas import tpu_sc as plsc

_NC, _NS = 2, 16
_NW = _NC * _NS
_CB = 80


def _sc_gather_rows(table, idx):
    V, D = table.shape
    B = idx.shape[0]
    CB = 128
    per_w = B // _NW
    n_chunks = per_w // CB
    idx3 = idx.reshape(_NW, n_chunks, CB)
    mesh = plsc.VectorSubcoreMesh(core_axis_name="c", subcore_axis_name="s")

    @functools.partial(
        pl.kernel,
        out_type=jax.ShapeDtypeStruct((B, D), table.dtype),
        mesh=mesh,
        scratch_types=[
            pltpu.VMEM((n_chunks, CB), jnp.int32),
            pltpu.VMEM((CB, D), table.dtype),
            pltpu.VMEM((CB, D), table.dtype),
            pltpu.SemaphoreType.DMA,
            pltpu.SemaphoreType.DMA,
        ],
    )
    def k(table_hbm, idx_hbm, out_hbm, idx_v, r0, r1, s0, s1):
        wid = lax.axis_index("s") * _NC + lax.axis_index("c")
        base = wid * per_w
        pltpu.sync_copy(idx_hbm.at[wid], idx_v)

        def start(c, buf, sem):
            return pltpu.async_copy(table_hbm.at[idx_v.at[c]], buf, sem)

        def wait(buf, sem):
            pltpu.make_async_copy(table_hbm.at[idx_v.at[0]], buf, sem).wait()

        start(0, r0, s0)

        @pl.loop(0, n_chunks, step=2)
        def _(c):
            @pl.when(c + 1 < n_chunks)
            def _():
                start(c + 1, r1, s1)

            wait(r0, s0)
            pltpu.sync_copy(r0, out_hbm.at[pl.ds(base + c * CB, CB)])

            @pl.when(c + 2 < n_chunks)
            def _():
                start(c + 2, r0, s0)

            @pl.when(c + 1 < n_chunks)
            def _():
                wait(r1, s1)
                pltpu.sync_copy(r1, out_hbm.at[pl.ds(base + (c + 1) * CB, CB)])

    return k(table, idx3)


def _sc_segment_sum(vals, idx, num_out, row_off, n_rows):
    D = vals.shape[1]
    Dh = D // _NC
    per_sub = n_rows // _NS
    n_chunks = per_sub // _CB
    ZR = 40
    n_zc = num_out // ZR
    idx3 = idx.reshape(_NS, n_chunks, _CB)
    mesh = plsc.VectorSubcoreMesh(core_axis_name="c", subcore_axis_name="s")

    @functools.partial(
        pl.kernel,
        out_type=jax.ShapeDtypeStruct((num_out, D), jnp.float32),
        mesh=mesh,
        scratch_types=[
            pltpu.VMEM((n_chunks, _CB), jnp.int32),
            pltpu.VMEM((40, Dh), jnp.float32),
            pltpu.VMEM_SHARED((num_out, Dh), jnp.float32),
            pltpu.VMEM((_CB, Dh), jnp.float32),
            pltpu.VMEM((_CB, Dh), jnp.float32),
            pltpu.SemaphoreType.DMA,
            pltpu.SemaphoreType.DMA,
        ],
    )
    def k(vals_hbm, idx_hbm, out_hbm, idx_v, zbuf, acc, r0, r1, s0, s1):
        cid = lax.axis_index("c")
        sid = lax.axis_index("s")
        col0 = cid * Dh

        @pl.loop(0, ZR)
        def _(r):
            @pl.loop(0, Dh, step=16)
            def _(j):
                zbuf[r, pl.ds(j, 16)] = jnp.zeros((16,), jnp.float32)

        @pl.loop(sid, n_zc, step=_NS)
        def _(zc):
            pltpu.sync_copy(zbuf, acc.at[pl.ds(zc * ZR, ZR)])

        plsc.subcore_barrier()

        pltpu.sync_copy(idx_hbm.at[sid], idx_v)
        base = row_off + sid * per_sub

        def start(c, buf, sem):
            pltpu.async_copy(
                vals_hbm.at[pl.ds(base + c * _CB, _CB), pl.ds(col0, Dh)],
                buf, sem)

        def wait(buf, sem):
            pltpu.make_async_copy(
                vals_hbm.at[pl.ds(base, _CB), pl.ds(col0, Dh)], buf,
                sem).wait()

        start(0, r0, s0)

        @pl.loop(0, n_chunks, step=2)
        def _(c):
            @pl.when(c + 1 < n_chunks)
            def _():
                start(c + 1, r1, s1)

            wait(r0, s0)
            pltpu.sync_copy(r0, acc.at[idx_v.at[c]], add=True)

            @pl.when(c + 2 < n_chunks)
            def _():
                start(c + 2, r0, s0)

            @pl.when(c + 1 < n_chunks)
            def _():
                wait(r1, s1)
                pltpu.sync_copy(r1, acc.at[idx_v.at[c + 1]], add=True)

        plsc.subcore_barrier()

        @pl.loop(sid, n_zc, step=_NS)
        def _(zc):
            pltpu.sync_copy(
                acc.at[pl.ds(zc * ZR, ZR)],
                out_hbm.at[pl.ds(zc * ZR, ZR), pl.ds(col0, Dh)])

    return k(vals, idx3)


def _edge_mlp(edges, gathered_i32, w1e, w1n_lo, w1n_hi, b1, w2, b2):
    E, DE = edges.shape
    G = gathered_i32.shape[1]
    H = w1e.shape[1]
    DO = w2.shape[1]
    BE = 1280
    nt = E // BE

    def body(e_ref, sn_ref, rn_ref, w1e_ref, w1lo_ref, w1hi_ref, b1_ref,
             w2_ref, b2_ref, out_ref):
        su = sn_ref[...]
        ru = rn_ref[...]
        mask = jnp.int32(-65536)
        s_lo = (lax.bitcast_convert_type(su << 16, jnp.float32)
                + lax.bitcast_convert_type(ru << 16, jnp.float32))
        s_hi = (lax.bitcast_convert_type(su & mask, jnp.float32)
                + lax.bitcast_convert_type(ru & mask, jnp.float32))
        x = jnp.dot(s_lo.astype(jnp.bfloat16), w1lo_ref[...],
                    preferred_element_type=jnp.float32)
        x += jnp.dot(s_hi.astype(jnp.bfloat16), w1hi_ref[...],
                     preferred_element_type=jnp.float32)
        x += jnp.dot(e_ref[...].astype(jnp.bfloat16), w1e_ref[...],
                     preferred_element_type=jnp.float32)
        x += b1_ref[...]
        h = jnp.maximum(x, 0.0).astype(jnp.bfloat16)
        y = jnp.dot(h, w2_ref[...], preferred_element_type=jnp.float32)
        y += b2_ref[...]
        out_ref[...] = y

    return pl.pallas_call(
        body,
        grid=(nt,),
        in_specs=[
            pl.BlockSpec((BE, DE), lambda i: (i, 0)),
            pl.BlockSpec((BE, G), lambda i: (i, 0)),
            pl.BlockSpec((BE, G), lambda i, n=nt: (n + i, 0)),
            pl.BlockSpec(w1e.shape, lambda i: (0, 0)),
            pl.BlockSpec(w1n_lo.shape, lambda i: (0, 0)),
            pl.BlockSpec(w1n_hi.shape, lambda i: (0, 0)),
            pl.BlockSpec((1, H), lambda i: (0, 0)),
            pl.BlockSpec(w2.shape, lambda i: (0, 0)),
            pl.BlockSpec((1, DO), lambda i: (0, 0)),
        ],
        out_specs=pl.BlockSpec((BE, DO), lambda i: (i, 0)),
        out_shape=jax.ShapeDtypeStruct((E, DO), jnp.float32),
        compiler_params=pltpu.CompilerParams(
            dimension_semantics=("parallel",)),
    )(edges, gathered_i32, gathered_i32, w1e, w1n_lo, w1n_hi, b1, w2, b2)


def _node_and_global_mlp(nodes, agg, w1n, w1a, b1, w2, b2,
                         g, inv_e, inv_n, wg1g, wg1e, wg1n, bg1_, wg2,
                         bg2_):
    N, DN = nodes.shape
    H = w1n.shape[1]
    DO = w2.shape[1]
    DGO = wg2.shape[1]
    DA = agg.shape[1]
    BN = 1000
    nt = N // BN

    def body(n_ref, a_ref, w1n_ref, w1a_ref, b1_ref, w2_ref, b2_ref,
             g_ref, wg1g_ref, wg1e_ref, wg1n_ref, bg1_ref, wg2_ref,
             bg2_ref, out_ref, go_ref, nsum_ref, esum_ref):
        i = pl.program_id(0)
        a = a_ref[...]
        x = jnp.dot(n_ref[...].astype(jnp.bfloat16), w1n_ref[...],
                    preferred_element_type=jnp.float32)
        x += jnp.dot(a.astype(jnp.bfloat16), w1a_ref[...],
                     preferred_element_type=jnp.float32)
        x += b1_ref[...]
        h = jnp.maximum(x, 0.0).astype(jnp.bfloat16)
        y = jnp.dot(h, w2_ref[...], preferred_element_type=jnp.float32)
        y += b2_ref[...]
        out_ref[...] = y
        part = jnp.sum(y, axis=0, keepdims=True)
        epart = jnp.sum(a, axis=0, keepdims=True)

        @pl.when(i == 0)
        def _():
            nsum_ref[...] = part
            esum_ref[...] = epart

        @pl.when(i > 0)
        def _():
            nsum_ref[...] += part
            esum_ref[...] += epart

        @pl.when(i == nt - 1)
        def _():
            gx = jnp.dot(g_ref[...], wg1g_ref[...],
                         preferred_element_type=jnp.float32)
            gx += jnp.dot(esum_ref[...] * inv_e, wg1e_ref[...],
                          preferred_element_type=jnp.float32)
            gx += jnp.dot(nsum_ref[...] * inv_n, wg1n_ref[...],
                          preferred_element_type=jnp.float32)
            gx += bg1_ref[...]
            gh = jnp.maximum(gx, 0.0)
            go_ref[...] = jnp.dot(gh, wg2_ref[...],
                                  preferred_element_type=jnp.float32)
            go_ref[...] += bg2_ref[...]

    full = lambda a: pl.BlockSpec(a.shape, lambda i: (0,) * a.ndim)
    return pl.pallas_call(
        body,
        grid=(nt,),
        in_specs=[
            pl.BlockSpec((BN, DN), lambda i: (i, 0)),
            pl.BlockSpec((BN, DA), lambda i: (i, 0)),
            full(w1n), full(w1a), pl.BlockSpec((1, H), lambda i: (0, 0)),
            full(w2), pl.BlockSpec((1, DO), lambda i: (0, 0)),
            full(g), full(wg1g), full(wg1e), full(wg1n),
            pl.BlockSpec((1, H), lambda i: (0, 0)), full(wg2),
            pl.BlockSpec((1, DGO), lambda i: (0, 0)),
        ],
        out_specs=[
            pl.BlockSpec((BN, DO), lambda i: (i, 0)),
            pl.BlockSpec((1, DGO), lambda i: (0, 0)),
            pl.BlockSpec((1, DO), lambda i: (0, 0)),
            pl.BlockSpec((1, DA), lambda i: (0, 0)),
        ],
        out_shape=[
            jax.ShapeDtypeStruct((N, DO), jnp.float32),
            jax.ShapeDtypeStruct((1, DGO), jnp.float32),
            jax.ShapeDtypeStruct((1, DO), jnp.float32),
            jax.ShapeDtypeStruct((1, DA), jnp.float32),
        ],
        compiler_params=pltpu.CompilerParams(
            dimension_semantics=("arbitrary",)),
    )(nodes, agg, w1n, w1a, b1, w2, b2, g, wg1g, wg1e, wg1n,
      bg1_.reshape(1, H), wg2, bg2_.reshape(1, DGO))


def kernel(nodes, edges, senders, receivers, globals_, We1, be1, We2, be2,
           Wn1, bn1, Wn2, bn2, Wg1, bg1, Wg2, bg2):
    N, DN = nodes.shape
    E, DE = edges.shape
    DG = globals_.shape[1]
    H = We1.shape[1]
    DE_OUT = We2.shape[1]
    DN_OUT = Wn2.shape[1]

    bf = jnp.bfloat16
    senders = senders.astype(jnp.int32)
    receivers = receivers.astype(jnp.int32)

    nodes_bf = nodes.astype(bf)
    Dh = DN // 2
    table_i32 = lax.bitcast_convert_type(
        jnp.stack([nodes_bf[:, :Dh], nodes_bf[:, Dh:]], axis=-1), jnp.int32)
    B_pad = ((2 * E) + (128 * _NW) - 1) // (128 * _NW) * (128 * _NW)
    idx_all = jnp.concatenate(
        [senders, receivers,
         jnp.zeros((B_pad - 2 * E,), jnp.int32)])
    gathered_i32 = _sc_gather_rows(table_i32, idx_all)

    w1e = We1[:DE].astype(bf)
    w1n_lo = (0.5 * We1[DE:DE + Dh]).astype(bf)
    w1n_hi = (0.5 * We1[DE + Dh:DE + DN]).astype(bf)
    b1e = (be1 + globals_[0] @ We1[DE + DN:]).reshape(1, H)
    new_edges = _edge_mlp(edges, gathered_i32, w1e, w1n_lo, w1n_hi,
                          b1e, We2.astype(bf), be2.reshape(1, DE_OUT))

    agg = _sc_segment_sum(new_edges, receivers, N, 0, E)

    wn1n = Wn1[:DN].astype(bf)
    wn1a = Wn1[DN:DN + DE_OUT].astype(bf)
    b1n = (bn1 + globals_[0] @ Wn1[DN + DE_OUT:]).reshape(1, H)
    wg1g = Wg1[:DG]
    wg1e = Wg1[DG:DG + DE_OUT]
    wg1n = Wg1[DG + DE_OUT:]
    new_nodes, new_globals, _, _ = _node_and_global_mlp(
        nodes, agg, wn1n, wn1a, b1n, Wn2.astype(bf), bn2.reshape(1, DN_OUT),
        globals_, 1.0 / E, 1.0 / N, wg1g, wg1e, wg1n, bg1, Wg2, bg2)

    return new_nodes, new_edges, new_globals


# speedup vs baseline: 1.3226x; 1.3226x over previous
import functools

import jax
import jax.numpy as jnp
from jax import lax
from jax.experimental import pallas as pl
from jax.experimental.pallas import tpu as pltpu
from jax.experimental.pallas import tpu_sc as plsc

_NC, _NS = 2, 16
_NW = _NC * _NS
_CB = 80


def _sc_gather_rows(table, idx):
    V, D = table.shape
    B = idx.shape[0]
    CB = _CB
    per_w = B // _NW
    n_chunks = per_w // CB
    idx3 = idx.reshape(_NW, n_chunks, CB)
    mesh = plsc.VectorSubcoreMesh(core_axis_name="c", subcore_axis_name="s")

    @functools.partial(
        pl.kernel,
        out_type=jax.ShapeDtypeStruct((B, D), table.dtype),
        mesh=mesh,
        scratch_types=[
            pltpu.VMEM((n_chunks, CB), jnp.int32),
            pltpu.VMEM((CB, D), table.dtype),
            pltpu.VMEM((CB, D), table.dtype),
            pltpu.SemaphoreType.DMA,
            pltpu.SemaphoreType.DMA,
        ],
    )
    def k(table_hbm, idx_hbm, out_hbm, idx_v, r0, r1, s0, s1):
        wid = lax.axis_index("s") * _NC + lax.axis_index("c")
        base = wid * per_w
        pltpu.sync_copy(idx_hbm.at[wid], idx_v)

        def start(c, buf, sem):
            return pltpu.async_copy(table_hbm.at[idx_v.at[c]], buf, sem)

        def wait(buf, sem):
            pltpu.make_async_copy(table_hbm.at[idx_v.at[0]], buf, sem).wait()

        start(0, r0, s0)

        @pl.loop(0, n_chunks, step=2)
        def _(c):
            @pl.when(c + 1 < n_chunks)
            def _():
                start(c + 1, r1, s1)

            wait(r0, s0)
            pltpu.sync_copy(r0, out_hbm.at[pl.ds(base + c * CB, CB)])

            @pl.when(c + 2 < n_chunks)
            def _():
                start(c + 2, r0, s0)

            @pl.when(c + 1 < n_chunks)
            def _():
                wait(r1, s1)
                pltpu.sync_copy(r1, out_hbm.at[pl.ds(base + (c + 1) * CB, CB)])

    return k(table, idx3)


def _sc_segment_sum(vals, idx, num_out, row_off, n_rows):
    D = vals.shape[1]
    Dh = D // _NC
    per_sub = n_rows // _NS
    n_chunks = per_sub // _CB
    ZR = 40
    n_zc = num_out // ZR
    idx3 = idx.reshape(_NS, n_chunks, _CB)
    mesh = plsc.VectorSubcoreMesh(core_axis_name="c", subcore_axis_name="s")

    @functools.partial(
        pl.kernel,
        out_type=jax.ShapeDtypeStruct((num_out, D), jnp.float32),
        mesh=mesh,
        scratch_types=[
            pltpu.VMEM((n_chunks, _CB), jnp.int32),
            pltpu.VMEM((40, Dh), jnp.float32),
            pltpu.VMEM_SHARED((num_out, Dh), jnp.float32),
            pltpu.VMEM((_CB, Dh), jnp.float32),
            pltpu.VMEM((_CB, Dh), jnp.float32),
            pltpu.SemaphoreType.DMA,
            pltpu.SemaphoreType.DMA,
        ],
    )
    def k(vals_hbm, idx_hbm, out_hbm, idx_v, zbuf, acc, r0, r1, s0, s1):
        cid = lax.axis_index("c")
        sid = lax.axis_index("s")
        col0 = cid * Dh

        @pl.loop(0, ZR)
        def _(r):
            @pl.loop(0, Dh, step=16)
            def _(j):
                zbuf[r, pl.ds(j, 16)] = jnp.zeros((16,), jnp.float32)

        @pl.loop(sid, n_zc, step=_NS)
        def _(zc):
            pltpu.sync_copy(zbuf, acc.at[pl.ds(zc * ZR, ZR)])

        plsc.subcore_barrier()

        pltpu.sync_copy(idx_hbm.at[sid], idx_v)
        base = row_off + sid * per_sub

        def start(c, buf, sem):
            pltpu.async_copy(
                vals_hbm.at[pl.ds(base + c * _CB, _CB), pl.ds(col0, Dh)],
                buf, sem)

        def wait(buf, sem):
            pltpu.make_async_copy(
                vals_hbm.at[pl.ds(base, _CB), pl.ds(col0, Dh)], buf,
                sem).wait()

        start(0, r0, s0)

        @pl.loop(0, n_chunks, step=2)
        def _(c):
            @pl.when(c + 1 < n_chunks)
            def _():
                start(c + 1, r1, s1)

            wait(r0, s0)
            pltpu.sync_copy(r0, acc.at[idx_v.at[c]], add=True)

            @pl.when(c + 2 < n_chunks)
            def _():
                start(c + 2, r0, s0)

            @pl.when(c + 1 < n_chunks)
            def _():
                wait(r1, s1)
                pltpu.sync_copy(r1, acc.at[idx_v.at[c + 1]], add=True)

        plsc.subcore_barrier()

        @pl.loop(sid, n_zc, step=_NS)
        def _(zc):
            pltpu.sync_copy(
                acc.at[pl.ds(zc * ZR, ZR)],
                out_hbm.at[pl.ds(zc * ZR, ZR), pl.ds(col0, Dh)])

    return k(vals, idx3)


def _edge_mlp(edges, gathered_i32, w1e, w1n_lo, w1n_hi, b1, w2, b2):
    E, DE = edges.shape
    G = gathered_i32.shape[1]
    H = w1e.shape[1]
    DO = w2.shape[1]
    BE = 1280
    nt = E // BE

    def body(e_ref, sn_ref, rn_ref, w1e_ref, w1lo_ref, w1hi_ref, b1_ref,
             w2_ref, b2_ref, out_ref):
        su = sn_ref[...]
        ru = rn_ref[...]
        mask = jnp.int32(-65536)
        s_lo = (lax.bitcast_convert_type(su << 16, jnp.float32)
                + lax.bitcast_convert_type(ru << 16, jnp.float32))
        s_hi = (lax.bitcast_convert_type(su & mask, jnp.float32)
                + lax.bitcast_convert_type(ru & mask, jnp.float32))
        x = jnp.dot(s_lo.astype(jnp.bfloat16), w1lo_ref[...],
                    preferred_element_type=jnp.float32)
        x += jnp.dot(s_hi.astype(jnp.bfloat16), w1hi_ref[...],
                     preferred_element_type=jnp.float32)
        x += jnp.dot(e_ref[...].astype(jnp.bfloat16), w1e_ref[...],
                     preferred_element_type=jnp.float32)
        x += b1_ref[...]
        h = jnp.maximum(x, 0.0).astype(jnp.bfloat16)
        y = jnp.dot(h, w2_ref[...], preferred_element_type=jnp.float32)
        y += b2_ref[...]
        out_ref[...] = y

    return pl.pallas_call(
        body,
        grid=(nt,),
        in_specs=[
            pl.BlockSpec((BE, DE), lambda i: (i, 0)),
            pl.BlockSpec((BE, G), lambda i: (i, 0)),
            pl.BlockSpec((BE, G), lambda i, n=nt: (n + i, 0)),
            pl.BlockSpec(w1e.shape, lambda i: (0, 0)),
            pl.BlockSpec(w1n_lo.shape, lambda i: (0, 0)),
            pl.BlockSpec(w1n_hi.shape, lambda i: (0, 0)),
            pl.BlockSpec((1, H), lambda i: (0, 0)),
            pl.BlockSpec(w2.shape, lambda i: (0, 0)),
            pl.BlockSpec((1, DO), lambda i: (0, 0)),
        ],
        out_specs=pl.BlockSpec((BE, DO), lambda i: (i, 0)),
        out_shape=jax.ShapeDtypeStruct((E, DO), jnp.float32),
        compiler_params=pltpu.CompilerParams(
            dimension_semantics=("parallel",)),
    )(edges, gathered_i32, gathered_i32, w1e, w1n_lo, w1n_hi, b1, w2, b2)


def _node_and_global_mlp(nodes, agg, w1n, w1a, b1, w2, b2,
                         g, inv_e, inv_n, wg1g, wg1e, wg1n, bg1_, wg2,
                         bg2_):
    N, DN = nodes.shape
    H = w1n.shape[1]
    DO = w2.shape[1]
    DGO = wg2.shape[1]
    DA = agg.shape[1]
    BN = 1000
    nt = N // BN

    def body(n_ref, a_ref, w1n_ref, w1a_ref, b1_ref, w2_ref, b2_ref,
             g_ref, wg1g_ref, wg1e_ref, wg1n_ref, bg1_ref, wg2_ref,
             bg2_ref, out_ref, go_ref, nsum_ref, esum_ref):
        i = pl.program_id(0)
        a = a_ref[...]
        x = jnp.dot(n_ref[...].astype(jnp.bfloat16), w1n_ref[...],
                    preferred_element_type=jnp.float32)
        x += jnp.dot(a.astype(jnp.bfloat16), w1a_ref[...],
                     preferred_element_type=jnp.float32)
        x += b1_ref[...]
        h = jnp.maximum(x, 0.0).astype(jnp.bfloat16)
        y = jnp.dot(h, w2_ref[...], preferred_element_type=jnp.float32)
        y += b2_ref[...]
        out_ref[...] = y
        part = jnp.sum(y, axis=0, keepdims=True)
        epart = jnp.sum(a, axis=0, keepdims=True)

        @pl.when(i == 0)
        def _():
            nsum_ref[...] = part
            esum_ref[...] = epart

        @pl.when(i > 0)
        def _():
            nsum_ref[...] += part
            esum_ref[...] += epart

        @pl.when(i == nt - 1)
        def _():
            gx = jnp.dot(g_ref[...], wg1g_ref[...],
                         preferred_element_type=jnp.float32)
            gx += jnp.dot(esum_ref[...] * inv_e, wg1e_ref[...],
                          preferred_element_type=jnp.float32)
            gx += jnp.dot(nsum_ref[...] * inv_n, wg1n_ref[...],
                          preferred_element_type=jnp.float32)
            gx += bg1_ref[...]
            gh = jnp.maximum(gx, 0.0)
            go_ref[...] = jnp.dot(gh, wg2_ref[...],
                                  preferred_element_type=jnp.float32)
            go_ref[...] += bg2_ref[...]

    full = lambda a: pl.BlockSpec(a.shape, lambda i: (0,) * a.ndim)
    return pl.pallas_call(
        body,
        grid=(nt,),
        in_specs=[
            pl.BlockSpec((BN, DN), lambda i: (i, 0)),
            pl.BlockSpec((BN, DA), lambda i: (i, 0)),
            full(w1n), full(w1a), pl.BlockSpec((1, H), lambda i: (0, 0)),
            full(w2), pl.BlockSpec((1, DO), lambda i: (0, 0)),
            full(g), full(wg1g), full(wg1e), full(wg1n),
            pl.BlockSpec((1, H), lambda i: (0, 0)), full(wg2),
            pl.BlockSpec((1, DGO), lambda i: (0, 0)),
        ],
        out_specs=[
            pl.BlockSpec((BN, DO), lambda i: (i, 0)),
            pl.BlockSpec((1, DGO), lambda i: (0, 0)),
            pl.BlockSpec((1, DO), lambda i: (0, 0)),
            pl.BlockSpec((1, DA), lambda i: (0, 0)),
        ],
        out_shape=[
            jax.ShapeDtypeStruct((N, DO), jnp.float32),
            jax.ShapeDtypeStruct((1, DGO), jnp.float32),
            jax.ShapeDtypeStruct((1, DO), jnp.float32),
            jax.ShapeDtypeStruct((1, DA), jnp.float32),
        ],
        compiler_params=pltpu.CompilerParams(
            dimension_semantics=("arbitrary",)),
    )(nodes, agg, w1n, w1a, b1, w2, b2, g, wg1g, wg1e, wg1n,
      bg1_.reshape(1, H), wg2, bg2_.reshape(1, DGO))


def kernel(nodes, edges, senders, receivers, globals_, We1, be1, We2, be2,
           Wn1, bn1, Wn2, bn2, Wg1, bg1, Wg2, bg2):
    N, DN = nodes.shape
    E, DE = edges.shape
    DG = globals_.shape[1]
    H = We1.shape[1]
    DE_OUT = We2.shape[1]
    DN_OUT = Wn2.shape[1]

    bf = jnp.bfloat16
    senders = senders.astype(jnp.int32)
    receivers = receivers.astype(jnp.int32)

    nodes_bf = nodes.astype(bf)
    Dh = DN // 2
    table_i32 = lax.bitcast_convert_type(
        jnp.stack([nodes_bf[:, :Dh], nodes_bf[:, Dh:]], axis=-1), jnp.int32)
    idx_all = jnp.concatenate([senders, receivers])
    gathered_i32 = _sc_gather_rows(table_i32, idx_all)

    w1e = We1[:DE].astype(bf)
    w1n_lo = (0.5 * We1[DE:DE + Dh]).astype(bf)
    w1n_hi = (0.5 * We1[DE + Dh:DE + DN]).astype(bf)
    b1e = (be1 + globals_[0] @ We1[DE + DN:]).reshape(1, H)
    new_edges = _edge_mlp(edges, gathered_i32, w1e, w1n_lo, w1n_hi,
                          b1e, We2.astype(bf), be2.reshape(1, DE_OUT))

    agg = _sc_segment_sum(new_edges, receivers, N, 0, E)

    wn1n = Wn1[:DN].astype(bf)
    wn1a = Wn1[DN:DN + DE_OUT].astype(bf)
    b1n = (bn1 + globals_[0] @ Wn1[DN + DE_OUT:]).reshape(1, H)
    wg1g = Wg1[:DG]
    wg1e = Wg1[DG:DG + DE_OUT]
    wg1n = Wg1[DG + DE_OUT:]
    new_nodes, new_globals, _, _ = _node_and_global_mlp(
        nodes, agg, wn1n, wn1a, b1n, Wn2.astype(bf), bn2.reshape(1, DN_OUT),
        globals_, 1.0 / E, 1.0 / N, wg1g, wg1e, wg1n, bg1, Wg2, bg2)

    return new_nodes, new_edges, new_globals


# speedup vs baseline: 1.3972x; 1.0564x over previous
import functools

import jax
import jax.numpy as jnp
from jax import lax
from jax.experimental import pallas as pl
from jax.experimental.pallas import tpu as pltpu
from jax.experimental.pallas import tpu_sc as plsc

_NC, _NS = 2, 16
_NW = _NC * _NS
_CB = 80


def _sc_gather_rows(table, idx):
    V, D = table.shape
    B = idx.shape[0]
    CB = _CB
    per_w = B // _NW
    n_chunks = per_w // CB
    idx3 = idx.reshape(_NW, n_chunks, CB)
    mesh = plsc.VectorSubcoreMesh(core_axis_name="c", subcore_axis_name="s")

    @functools.partial(
        pl.kernel,
        out_type=jax.ShapeDtypeStruct((B, D), table.dtype),
        mesh=mesh,
        scratch_types=[
            pltpu.VMEM((n_chunks, CB), jnp.int32),
            pltpu.VMEM((CB, D), table.dtype),
            pltpu.VMEM((CB, D), table.dtype),
            pltpu.SemaphoreType.DMA,
            pltpu.SemaphoreType.DMA,
        ],
    )
    def k(table_hbm, idx_hbm, out_hbm, idx_v, r0, r1, s0, s1):
        wid = lax.axis_index("s") * _NC + lax.axis_index("c")
        base = wid * per_w
        pltpu.sync_copy(idx_hbm.at[wid], idx_v)

        def start(c, buf, sem):
            return pltpu.async_copy(table_hbm.at[idx_v.at[c]], buf, sem)

        def wait(buf, sem):
            pltpu.make_async_copy(table_hbm.at[idx_v.at[0]], buf, sem).wait()

        start(0, r0, s0)

        @pl.loop(0, n_chunks, step=2)
        def _(c):
            @pl.when(c + 1 < n_chunks)
            def _():
                start(c + 1, r1, s1)

            wait(r0, s0)
            pltpu.sync_copy(r0, out_hbm.at[pl.ds(base + c * CB, CB)])

            @pl.when(c + 2 < n_chunks)
            def _():
                start(c + 2, r0, s0)

            @pl.when(c + 1 < n_chunks)
            def _():
                wait(r1, s1)
                pltpu.sync_copy(r1, out_hbm.at[pl.ds(base + (c + 1) * CB, CB)])

    return k(table, idx3)


def _sc_segment_sum(vals, idx, num_out, row_off, n_rows):
    D = vals.shape[1]
    Dh = D // _NC
    per_sub = n_rows // _NS
    n_chunks = per_sub // _CB
    ZR = 40
    n_zc = num_out // ZR
    idx3 = idx.reshape(_NS, n_chunks, _CB)
    mesh = plsc.VectorSubcoreMesh(core_axis_name="c", subcore_axis_name="s")

    @functools.partial(
        pl.kernel,
        out_type=jax.ShapeDtypeStruct((num_out, D), jnp.float32),
        mesh=mesh,
        scratch_types=[
            pltpu.VMEM((n_chunks, _CB), jnp.int32),
            pltpu.VMEM((40, Dh), jnp.float32),
            pltpu.VMEM_SHARED((num_out, Dh), jnp.float32),
            pltpu.VMEM((_CB, Dh), jnp.float32),
            pltpu.VMEM((_CB, Dh), jnp.float32),
            pltpu.SemaphoreType.DMA,
            pltpu.SemaphoreType.DMA,
        ],
    )
    def k(vals_hbm, idx_hbm, out_hbm, idx_v, zbuf, acc, r0, r1, s0, s1):
        cid = lax.axis_index("c")
        sid = lax.axis_index("s")
        col0 = cid * Dh

        @pl.loop(0, ZR)
        def _(r):
            @pl.loop(0, Dh, step=16)
            def _(j):
                zbuf[r, pl.ds(j, 16)] = jnp.zeros((16,), jnp.float32)

        @pl.loop(sid, n_zc, step=_NS)
        def _(zc):
            pltpu.sync_copy(zbuf, acc.at[pl.ds(zc * ZR, ZR)])

        plsc.subcore_barrier()

        pltpu.sync_copy(idx_hbm.at[sid], idx_v)
        base = row_off + sid * per_sub

        def start(c, buf, sem):
            pltpu.async_copy(
                vals_hbm.at[pl.ds(base + c * _CB, _CB), pl.ds(col0, Dh)],
                buf, sem)

        def wait(buf, sem):
            pltpu.make_async_copy(
                vals_hbm.at[pl.ds(base, _CB), pl.ds(col0, Dh)], buf,
                sem).wait()

        start(0, r0, s0)

        @pl.loop(0, n_chunks, step=2)
        def _(c):
            @pl.when(c + 1 < n_chunks)
            def _():
                start(c + 1, r1, s1)

            wait(r0, s0)
            pltpu.sync_copy(r0, acc.at[idx_v.at[c]], add=True)

            @pl.when(c + 2 < n_chunks)
            def _():
                start(c + 2, r0, s0)

            @pl.when(c + 1 < n_chunks)
            def _():
                wait(r1, s1)
                pltpu.sync_copy(r1, acc.at[idx_v.at[c + 1]], add=True)

        plsc.subcore_barrier()

        @pl.loop(sid, n_zc, step=_NS)
        def _(zc):
            pltpu.sync_copy(
                acc.at[pl.ds(zc * ZR, ZR)],
                out_hbm.at[pl.ds(zc * ZR, ZR), pl.ds(col0, Dh)])

    return k(vals, idx3)


def _edge_mlp(edges, gathered_i32, w1e, w1n_lo, w1n_hi, b1, w2, b2):
    E, DE = edges.shape
    G = gathered_i32.shape[1]
    H = w1e.shape[1]
    DO = w2.shape[1]
    BE = 2000
    nt = E // BE

    def body(e_ref, sn_ref, rn_ref, w1e_ref, w1lo_ref, w1hi_ref, b1_ref,
             w2_ref, b2_ref, out_ref):
        su = sn_ref[...]
        ru = rn_ref[...]
        mask = jnp.int32(-65536)
        s_lo = (lax.bitcast_convert_type(su << 16, jnp.float32)
                + lax.bitcast_convert_type(ru << 16, jnp.float32))
        s_hi = (lax.bitcast_convert_type(su & mask, jnp.float32)
                + lax.bitcast_convert_type(ru & mask, jnp.float32))
        x = jnp.dot(s_lo.astype(jnp.bfloat16), w1lo_ref[...],
                    preferred_element_type=jnp.float32)
        x += jnp.dot(s_hi.astype(jnp.bfloat16), w1hi_ref[...],
                     preferred_element_type=jnp.float32)
        x += jnp.dot(e_ref[...].astype(jnp.bfloat16), w1e_ref[...],
                     preferred_element_type=jnp.float32)
        x += b1_ref[...]
        h = jnp.maximum(x, 0.0).astype(jnp.bfloat16)
        y = jnp.dot(h, w2_ref[...], preferred_element_type=jnp.float32)
        y += b2_ref[...]
        out_ref[...] = y

    return pl.pallas_call(
        body,
        grid=(nt,),
        in_specs=[
            pl.BlockSpec((BE, DE), lambda i: (i, 0)),
            pl.BlockSpec((BE, G), lambda i: (i, 0)),
            pl.BlockSpec((BE, G), lambda i, n=nt: (n + i, 0)),
            pl.BlockSpec(w1e.shape, lambda i: (0, 0)),
            pl.BlockSpec(w1n_lo.shape, lambda i: (0, 0)),
            pl.BlockSpec(w1n_hi.shape, lambda i: (0, 0)),
            pl.BlockSpec((1, H), lambda i: (0, 0)),
            pl.BlockSpec(w2.shape, lambda i: (0, 0)),
            pl.BlockSpec((1, DO), lambda i: (0, 0)),
        ],
        out_specs=pl.BlockSpec((BE, DO), lambda i: (i, 0)),
        out_shape=jax.ShapeDtypeStruct((E, DO), jnp.float32),
        compiler_params=pltpu.CompilerParams(
            dimension_semantics=("parallel",)),
    )(edges, gathered_i32, gathered_i32, w1e, w1n_lo, w1n_hi, b1, w2, b2)


def _node_and_global_mlp(nodes, agg, w1n, w1a, b1, w2, b2,
                         g, inv_e, inv_n, wg1g, wg1e, wg1n, bg1_, wg2,
                         bg2_):
    N, DN = nodes.shape
    H = w1n.shape[1]
    DO = w2.shape[1]
    DGO = wg2.shape[1]
    DA = agg.shape[1]
    BN = 1000
    nt = N // BN

    def body(n_ref, a_ref, w1n_ref, w1a_ref, b1_ref, w2_ref, b2_ref,
             g_ref, wg1g_ref, wg1e_ref, wg1n_ref, bg1_ref, wg2_ref,
             bg2_ref, out_ref, go_ref, nsum_ref, esum_ref):
        i = pl.program_id(0)
        a = a_ref[...]
        x = jnp.dot(n_ref[...].astype(jnp.bfloat16), w1n_ref[...],
                    preferred_element_type=jnp.float32)
        x += jnp.dot(a.astype(jnp.bfloat16), w1a_ref[...],
                     preferred_element_type=jnp.float32)
        x += b1_ref[...]
        h = jnp.maximum(x, 0.0).astype(jnp.bfloat16)
        y = jnp.dot(h, w2_ref[...], preferred_element_type=jnp.float32)
        y += b2_ref[...]
        out_ref[...] = y
        part = jnp.sum(y, axis=0, keepdims=True)
        epart = jnp.sum(a, axis=0, keepdims=True)

        @pl.when(i == 0)
        def _():
            nsum_ref[...] = part
            esum_ref[...] = epart

        @pl.when(i > 0)
        def _():
            nsum_ref[...] += part
            esum_ref[...] += epart

        @pl.when(i == nt - 1)
        def _():
            gx = jnp.dot(g_ref[...], wg1g_ref[...],
                         preferred_element_type=jnp.float32)
            gx += jnp.dot(esum_ref[...] * inv_e, wg1e_ref[...],
                          preferred_element_type=jnp.float32)
            gx += jnp.dot(nsum_ref[...] * inv_n, wg1n_ref[...],
                          preferred_element_type=jnp.float32)
            gx += bg1_ref[...]
            gh = jnp.maximum(gx, 0.0)
            go_ref[...] = jnp.dot(gh, wg2_ref[...],
                                  preferred_element_type=jnp.float32)
            go_ref[...] += bg2_ref[...]

    full = lambda a: pl.BlockSpec(a.shape, lambda i: (0,) * a.ndim)
    return pl.pallas_call(
        body,
        grid=(nt,),
        in_specs=[
            pl.BlockSpec((BN, DN), lambda i: (i, 0)),
            pl.BlockSpec((BN, DA), lambda i: (i, 0)),
            full(w1n), full(w1a), pl.BlockSpec((1, H), lambda i: (0, 0)),
            full(w2), pl.BlockSpec((1, DO), lambda i: (0, 0)),
            full(g), full(wg1g), full(wg1e), full(wg1n),
            pl.BlockSpec((1, H), lambda i: (0, 0)), full(wg2),
            pl.BlockSpec((1, DGO), lambda i: (0, 0)),
        ],
        out_specs=[
            pl.BlockSpec((BN, DO), lambda i: (i, 0)),
            pl.BlockSpec((1, DGO), lambda i: (0, 0)),
            pl.BlockSpec((1, DO), lambda i: (0, 0)),
            pl.BlockSpec((1, DA), lambda i: (0, 0)),
        ],
        out_shape=[
            jax.ShapeDtypeStruct((N, DO), jnp.float32),
            jax.ShapeDtypeStruct((1, DGO), jnp.float32),
            jax.ShapeDtypeStruct((1, DO), jnp.float32),
            jax.ShapeDtypeStruct((1, DA), jnp.float32),
        ],
        compiler_params=pltpu.CompilerParams(
            dimension_semantics=("arbitrary",)),
    )(nodes, agg, w1n, w1a, b1, w2, b2, g, wg1g, wg1e, wg1n,
      bg1_.reshape(1, H), wg2, bg2_.reshape(1, DGO))


def kernel(nodes, edges, senders, receivers, globals_, We1, be1, We2, be2,
           Wn1, bn1, Wn2, bn2, Wg1, bg1, Wg2, bg2):
    N, DN = nodes.shape
    E, DE = edges.shape
    DG = globals_.shape[1]
    H = We1.shape[1]
    DE_OUT = We2.shape[1]
    DN_OUT = Wn2.shape[1]

    bf = jnp.bfloat16
    senders = senders.astype(jnp.int32)
    receivers = receivers.astype(jnp.int32)

    nodes_bf = nodes.astype(bf)
    Dh = DN // 2
    table_i32 = lax.bitcast_convert_type(
        jnp.stack([nodes_bf[:, :Dh], nodes_bf[:, Dh:]], axis=-1), jnp.int32)
    idx_all = jnp.concatenate([senders, receivers])
    gathered_i32 = _sc_gather_rows(table_i32, idx_all)

    w1e = We1[:DE].astype(bf)
    w1n_lo = (0.5 * We1[DE:DE + Dh]).astype(bf)
    w1n_hi = (0.5 * We1[DE + Dh:DE + DN]).astype(bf)
    b1e = (be1 + globals_[0] @ We1[DE + DN:]).reshape(1, H)
    new_edges = _edge_mlp(edges, gathered_i32, w1e, w1n_lo, w1n_hi,
                          b1e, We2.astype(bf), be2.reshape(1, DE_OUT))

    agg = _sc_segment_sum(new_edges, receivers, N, 0, E)

    wn1n = Wn1[:DN].astype(bf)
    wn1a = Wn1[DN:DN + DE_OUT].astype(bf)
    b1n = (bn1 + globals_[0] @ Wn1[DN + DE_OUT:]).reshape(1, H)
    wg1g = Wg1[:DG]
    wg1e = Wg1[DG:DG + DE_OUT]
    wg1n = Wg1[DG + DE_OUT:]
    new_nodes, new_globals, _, _ = _node_and_global_mlp(
        nodes, agg, wn1n, wn1a, b1n, Wn2.astype(bf), bn2.reshape(1, DN_OUT),
        globals_, 1.0 / E, 1.0 / N, wg1g, wg1e, wg1n, bg1, Wg2, bg2)

    return new_nodes, new_edges, new_globals


# speedup vs baseline: 1.4562x; 1.0422x over previous
import functools

import jax
import jax.numpy as jnp
from jax import lax
from jax.experimental import pallas as pl
from jax.experimental.pallas import tpu as pltpu
from jax.experimental.pallas import tpu_sc as plsc

_NC, _NS = 2, 16
_NW = _NC * _NS
_CB = 80


def _sc_gather_rows(table, idx):
    V, D = table.shape
    B = idx.shape[0]
    CB = _CB
    per_w = B // _NW
    n_chunks = per_w // CB
    idx3 = idx.reshape(_NW, n_chunks, CB)
    mesh = plsc.VectorSubcoreMesh(core_axis_name="c", subcore_axis_name="s")

    @functools.partial(
        pl.kernel,
        out_type=jax.ShapeDtypeStruct((B, D), table.dtype),
        mesh=mesh,
        scratch_types=[
            pltpu.VMEM((n_chunks, CB), jnp.int32),
            pltpu.VMEM((CB, D), table.dtype),
            pltpu.VMEM((CB, D), table.dtype),
            pltpu.SemaphoreType.DMA,
            pltpu.SemaphoreType.DMA,
        ],
    )
    def k(table_hbm, idx_hbm, out_hbm, idx_v, r0, r1, s0, s1):
        wid = lax.axis_index("s") * _NC + lax.axis_index("c")
        base = wid * per_w
        pltpu.sync_copy(idx_hbm.at[wid], idx_v)

        def start(c, buf, sem):
            return pltpu.async_copy(table_hbm.at[idx_v.at[c]], buf, sem)

        def wait(buf, sem):
            pltpu.make_async_copy(table_hbm.at[idx_v.at[0]], buf, sem).wait()

        start(0, r0, s0)

        @pl.loop(0, n_chunks, step=2)
        def _(c):
            @pl.when(c + 1 < n_chunks)
            def _():
                start(c + 1, r1, s1)

            wait(r0, s0)
            pltpu.sync_copy(r0, out_hbm.at[pl.ds(base + c * CB, CB)])

            @pl.when(c + 2 < n_chunks)
            def _():
                start(c + 2, r0, s0)

            @pl.when(c + 1 < n_chunks)
            def _():
                wait(r1, s1)
                pltpu.sync_copy(r1, out_hbm.at[pl.ds(base + (c + 1) * CB, CB)])

    return k(table, idx3)


def _sc_segment_sum(vals, idx, num_out, row_off, n_rows):
    D = vals.shape[1]
    Dh = D // _NC
    per_sub = n_rows // _NS
    n_chunks = per_sub // _CB
    ZR = 40
    n_zc = num_out // ZR
    idx3 = idx.reshape(_NS, n_chunks, _CB)
    mesh = plsc.VectorSubcoreMesh(core_axis_name="c", subcore_axis_name="s")

    @functools.partial(
        pl.kernel,
        out_type=jax.ShapeDtypeStruct((num_out, D), jnp.float32),
        mesh=mesh,
        scratch_types=[
            pltpu.VMEM((n_chunks, _CB), jnp.int32),
            pltpu.VMEM((40, Dh), jnp.float32),
            pltpu.VMEM_SHARED((num_out, Dh), jnp.float32),
            pltpu.VMEM((_CB, Dh), jnp.float32),
            pltpu.VMEM((_CB, Dh), jnp.float32),
            pltpu.SemaphoreType.DMA,
            pltpu.SemaphoreType.DMA,
        ],
    )
    def k(vals_hbm, idx_hbm, out_hbm, idx_v, zbuf, acc, r0, r1, s0, s1):
        cid = lax.axis_index("c")
        sid = lax.axis_index("s")
        col0 = cid * Dh

        @pl.loop(0, ZR)
        def _(r):
            @pl.loop(0, Dh, step=16)
            def _(j):
                zbuf[r, pl.ds(j, 16)] = jnp.zeros((16,), jnp.float32)

        @pl.loop(sid, n_zc, step=_NS)
        def _(zc):
            pltpu.sync_copy(zbuf, acc.at[pl.ds(zc * ZR, ZR)])

        plsc.subcore_barrier()

        pltpu.sync_copy(idx_hbm.at[sid], idx_v)
        base = row_off + sid * per_sub

        def start(c, buf, sem):
            pltpu.async_copy(
                vals_hbm.at[pl.ds(base + c * _CB, _CB), pl.ds(col0, Dh)],
                buf, sem)

        def wait(buf, sem):
            pltpu.make_async_copy(
                vals_hbm.at[pl.ds(base, _CB), pl.ds(col0, Dh)], buf,
                sem).wait()

        start(0, r0, s0)

        @pl.loop(0, n_chunks, step=2)
        def _(c):
            @pl.when(c + 1 < n_chunks)
            def _():
                start(c + 1, r1, s1)

            wait(r0, s0)
            pltpu.sync_copy(r0, acc.at[idx_v.at[c]], add=True)

            @pl.when(c + 2 < n_chunks)
            def _():
                start(c + 2, r0, s0)

            @pl.when(c + 1 < n_chunks)
            def _():
                wait(r1, s1)
                pltpu.sync_copy(r1, acc.at[idx_v.at[c + 1]], add=True)

        plsc.subcore_barrier()

        @pl.loop(sid, n_zc, step=_NS)
        def _(zc):
            pltpu.sync_copy(
                acc.at[pl.ds(zc * ZR, ZR)],
                out_hbm.at[pl.ds(zc * ZR, ZR), pl.ds(col0, Dh)])

    return k(vals, idx3)


def _edge_mlp(edges, gathered_i32, w1e, w1n_lo, w1n_hi, b1, w2, b2):
    E, DE = edges.shape
    G = gathered_i32.shape[1]
    H = w1e.shape[1]
    DO = w2.shape[1]
    BE = 4000
    nt = E // BE

    def body(e_ref, sn_ref, rn_ref, w1e_ref, w1lo_ref, w1hi_ref, b1_ref,
             w2_ref, b2_ref, out_ref):
        su = sn_ref[...]
        ru = rn_ref[...]
        mask = jnp.int32(-65536)
        s_lo = (lax.bitcast_convert_type(su << 16, jnp.float32)
                + lax.bitcast_convert_type(ru << 16, jnp.float32))
        s_hi = (lax.bitcast_convert_type(su & mask, jnp.float32)
                + lax.bitcast_convert_type(ru & mask, jnp.float32))
        x = jnp.dot(s_lo.astype(jnp.bfloat16), w1lo_ref[...],
                    preferred_element_type=jnp.float32)
        x += jnp.dot(s_hi.astype(jnp.bfloat16), w1hi_ref[...],
                     preferred_element_type=jnp.float32)
        x += jnp.dot(e_ref[...].astype(jnp.bfloat16), w1e_ref[...],
                     preferred_element_type=jnp.float32)
        x += b1_ref[...]
        h = jnp.maximum(x, 0.0).astype(jnp.bfloat16)
        y = jnp.dot(h, w2_ref[...], preferred_element_type=jnp.float32)
        y += b2_ref[...]
        out_ref[...] = y

    return pl.pallas_call(
        body,
        grid=(nt,),
        in_specs=[
            pl.BlockSpec((BE, DE), lambda i: (i, 0)),
            pl.BlockSpec((BE, G), lambda i: (i, 0)),
            pl.BlockSpec((BE, G), lambda i, n=nt: (n + i, 0)),
            pl.BlockSpec(w1e.shape, lambda i: (0, 0)),
            pl.BlockSpec(w1n_lo.shape, lambda i: (0, 0)),
            pl.BlockSpec(w1n_hi.shape, lambda i: (0, 0)),
            pl.BlockSpec((1, H), lambda i: (0, 0)),
            pl.BlockSpec(w2.shape, lambda i: (0, 0)),
            pl.BlockSpec((1, DO), lambda i: (0, 0)),
        ],
        out_specs=pl.BlockSpec((BE, DO), lambda i: (i, 0)),
        out_shape=jax.ShapeDtypeStruct((E, DO), jnp.float32),
        compiler_params=pltpu.CompilerParams(
            dimension_semantics=("parallel",)),
    )(edges, gathered_i32, gathered_i32, w1e, w1n_lo, w1n_hi, b1, w2, b2)


def _node_and_global_mlp(nodes, agg, w1n, w1a, b1, w2, b2,
                         g, inv_e, inv_n, wg1g, wg1e, wg1n, bg1_, wg2,
                         bg2_):
    N, DN = nodes.shape
    H = w1n.shape[1]
    DO = w2.shape[1]
    DGO = wg2.shape[1]
    DA = agg.shape[1]
    BN = 1000
    nt = N // BN

    def body(n_ref, a_ref, w1n_ref, w1a_ref, b1_ref, w2_ref, b2_ref,
             g_ref, wg1g_ref, wg1e_ref, wg1n_ref, bg1_ref, wg2_ref,
             bg2_ref, out_ref, go_ref, nsum_ref, esum_ref):
        i = pl.program_id(0)
        a = a_ref[...]
        x = jnp.dot(n_ref[...].astype(jnp.bfloat16), w1n_ref[...],
                    preferred_element_type=jnp.float32)
        x += jnp.dot(a.astype(jnp.bfloat16), w1a_ref[...],
                     preferred_element_type=jnp.float32)
        x += b1_ref[...]
        h = jnp.maximum(x, 0.0).astype(jnp.bfloat16)
        y = jnp.dot(h, w2_ref[...], preferred_element_type=jnp.float32)
        y += b2_ref[...]
        out_ref[...] = y
        part = jnp.sum(y, axis=0, keepdims=True)
        epart = jnp.sum(a, axis=0, keepdims=True)

        @pl.when(i == 0)
        def _():
            nsum_ref[...] = part
            esum_ref[...] = epart

        @pl.when(i > 0)
        def _():
            nsum_ref[...] += part
            esum_ref[...] += epart

        @pl.when(i == nt - 1)
        def _():
            gx = jnp.dot(g_ref[...], wg1g_ref[...],
                         preferred_element_type=jnp.float32)
            gx += jnp.dot(esum_ref[...] * inv_e, wg1e_ref[...],
                          preferred_element_type=jnp.float32)
            gx += jnp.dot(nsum_ref[...] * inv_n, wg1n_ref[...],
                          preferred_element_type=jnp.float32)
            gx += bg1_ref[...]
            gh = jnp.maximum(gx, 0.0)
            go_ref[...] = jnp.dot(gh, wg2_ref[...],
                                  preferred_element_type=jnp.float32)
            go_ref[...] += bg2_ref[...]

    full = lambda a: pl.BlockSpec(a.shape, lambda i: (0,) * a.ndim)
    return pl.pallas_call(
        body,
        grid=(nt,),
        in_specs=[
            pl.BlockSpec((BN, DN), lambda i: (i, 0)),
            pl.BlockSpec((BN, DA), lambda i: (i, 0)),
            full(w1n), full(w1a), pl.BlockSpec((1, H), lambda i: (0, 0)),
            full(w2), pl.BlockSpec((1, DO), lambda i: (0, 0)),
            full(g), full(wg1g), full(wg1e), full(wg1n),
            pl.BlockSpec((1, H), lambda i: (0, 0)), full(wg2),
            pl.BlockSpec((1, DGO), lambda i: (0, 0)),
        ],
        out_specs=[
            pl.BlockSpec((BN, DO), lambda i: (i, 0)),
            pl.BlockSpec((1, DGO), lambda i: (0, 0)),
            pl.BlockSpec((1, DO), lambda i: (0, 0)),
            pl.BlockSpec((1, DA), lambda i: (0, 0)),
        ],
        out_shape=[
            jax.ShapeDtypeStruct((N, DO), jnp.float32),
            jax.ShapeDtypeStruct((1, DGO), jnp.float32),
            jax.ShapeDtypeStruct((1, DO), jnp.float32),
            jax.ShapeDtypeStruct((1, DA), jnp.float32),
        ],
        compiler_params=pltpu.CompilerParams(
            dimension_semantics=("arbitrary",)),
    )(nodes, agg, w1n, w1a, b1, w2, b2, g, wg1g, wg1e, wg1n,
      bg1_.reshape(1, H), wg2, bg2_.reshape(1, DGO))


def kernel(nodes, edges, senders, receivers, globals_, We1, be1, We2, be2,
           Wn1, bn1, Wn2, bn2, Wg1, bg1, Wg2, bg2):
    N, DN = nodes.shape
    E, DE = edges.shape
    DG = globals_.shape[1]
    H = We1.shape[1]
    DE_OUT = We2.shape[1]
    DN_OUT = Wn2.shape[1]

    bf = jnp.bfloat16
    senders = senders.astype(jnp.int32)
    receivers = receivers.astype(jnp.int32)

    nodes_bf = nodes.astype(bf)
    Dh = DN // 2
    table_i32 = lax.bitcast_convert_type(
        jnp.stack([nodes_bf[:, :Dh], nodes_bf[:, Dh:]], axis=-1), jnp.int32)
    idx_all = jnp.concatenate([senders, receivers])
    gathered_i32 = _sc_gather_rows(table_i32, idx_all)

    w1e = We1[:DE].astype(bf)
    w1n_lo = (0.5 * We1[DE:DE + Dh]).astype(bf)
    w1n_hi = (0.5 * We1[DE + Dh:DE + DN]).astype(bf)
    b1e = (be1 + globals_[0] @ We1[DE + DN:]).reshape(1, H)
    new_edges = _edge_mlp(edges, gathered_i32, w1e, w1n_lo, w1n_hi,
                          b1e, We2.astype(bf), be2.reshape(1, DE_OUT))

    agg = _sc_segment_sum(new_edges, receivers, N, 0, E)

    wn1n = Wn1[:DN].astype(bf)
    wn1a = Wn1[DN:DN + DE_OUT].astype(bf)
    b1n = (bn1 + globals_[0] @ Wn1[DN + DE_OUT:]).reshape(1, H)
    wg1g = Wg1[:DG]
    wg1e = Wg1[DG:DG + DE_OUT]
    wg1n = Wg1[DG + DE_OUT:]
    new_nodes, new_globals, _, _ = _node_and_global_mlp(
        nodes, agg, wn1n, wn1a, b1n, Wn2.astype(bf), bn2.reshape(1, DN_OUT),
        globals_, 1.0 / E, 1.0 / N, wg1g, wg1e, wg1n, bg1, Wg2, bg2)

    return new_nodes, new_edges, new_globals


# speedup vs baseline: 1.4685x; 1.0085x over previous
import functools

import jax
import jax.numpy as jnp
from jax import lax
from jax.experimental import pallas as pl
from jax.experimental.pallas import tpu as pltpu
from jax.experimental.pallas import tpu_sc as plsc

_NC, _NS = 2, 16
_NW = _NC * _NS
_CB = 80


def _sc_gather_rows(table, idx):
    V, D = table.shape
    B = idx.shape[0]
    CB = _CB
    per_w = B // _NW
    n_chunks = per_w // CB
    idx3 = idx.reshape(_NW, n_chunks, CB)
    mesh = plsc.VectorSubcoreMesh(core_axis_name="c", subcore_axis_name="s")

    @functools.partial(
        pl.kernel,
        out_type=jax.ShapeDtypeStruct((B, D), table.dtype),
        mesh=mesh,
        scratch_types=[
            pltpu.VMEM((n_chunks, CB), jnp.int32),
            pltpu.VMEM((CB, D), table.dtype),
            pltpu.VMEM((CB, D), table.dtype),
            pltpu.SemaphoreType.DMA,
            pltpu.SemaphoreType.DMA,
        ],
    )
    def k(table_hbm, idx_hbm, out_hbm, idx_v, r0, r1, s0, s1):
        wid = lax.axis_index("s") * _NC + lax.axis_index("c")
        base = wid * per_w
        pltpu.sync_copy(idx_hbm.at[wid], idx_v)

        def start(c, buf, sem):
            return pltpu.async_copy(table_hbm.at[idx_v.at[c]], buf, sem)

        def wait(buf, sem):
            pltpu.make_async_copy(table_hbm.at[idx_v.at[0]], buf, sem).wait()

        start(0, r0, s0)

        @pl.loop(0, n_chunks, step=2)
        def _(c):
            @pl.when(c + 1 < n_chunks)
            def _():
                start(c + 1, r1, s1)

            wait(r0, s0)
            pltpu.sync_copy(r0, out_hbm.at[pl.ds(base + c * CB, CB)])

            @pl.when(c + 2 < n_chunks)
            def _():
                start(c + 2, r0, s0)

            @pl.when(c + 1 < n_chunks)
            def _():
                wait(r1, s1)
                pltpu.sync_copy(r1, out_hbm.at[pl.ds(base + (c + 1) * CB, CB)])

    return k(table, idx3)


def _sc_segment_sum(vals, idx, num_out, row_off, n_rows):
    D = vals.shape[1]
    Dh = D // _NC
    per_sub = n_rows // _NS
    n_chunks = per_sub // _CB
    ZR = 40
    n_zc = num_out // ZR
    idx3 = idx.reshape(_NS, n_chunks, _CB)
    mesh = plsc.VectorSubcoreMesh(core_axis_name="c", subcore_axis_name="s")

    @functools.partial(
        pl.kernel,
        out_type=jax.ShapeDtypeStruct((num_out, D), jnp.float32),
        mesh=mesh,
        scratch_types=[
            pltpu.VMEM((n_chunks, _CB), jnp.int32),
            pltpu.VMEM((40, Dh), jnp.float32),
            pltpu.VMEM_SHARED((num_out, Dh), jnp.float32),
            pltpu.VMEM((_CB, Dh), jnp.float32),
            pltpu.VMEM((_CB, Dh), jnp.float32),
            pltpu.SemaphoreType.DMA,
            pltpu.SemaphoreType.DMA,
        ],
    )
    def k(vals_hbm, idx_hbm, out_hbm, idx_v, zbuf, acc, r0, r1, s0, s1):
        cid = lax.axis_index("c")
        sid = lax.axis_index("s")
        col0 = cid * Dh

        @pl.loop(0, ZR)
        def _(r):
            @pl.loop(0, Dh, step=16)
            def _(j):
                zbuf[r, pl.ds(j, 16)] = jnp.zeros((16,), jnp.float32)

        @pl.loop(sid, n_zc, step=_NS)
        def _(zc):
            pltpu.sync_copy(zbuf, acc.at[pl.ds(zc * ZR, ZR)])

        plsc.subcore_barrier()

        pltpu.sync_copy(idx_hbm.at[sid], idx_v)
        base = row_off + sid * per_sub

        def start(c, buf, sem):
            pltpu.async_copy(
                vals_hbm.at[pl.ds(base + c * _CB, _CB), pl.ds(col0, Dh)],
                buf, sem)

        def wait(buf, sem):
            pltpu.make_async_copy(
                vals_hbm.at[pl.ds(base, _CB), pl.ds(col0, Dh)], buf,
                sem).wait()

        start(0, r0, s0)

        @pl.loop(0, n_chunks, step=2)
        def _(c):
            @pl.when(c + 1 < n_chunks)
            def _():
                start(c + 1, r1, s1)

            wait(r0, s0)
            pltpu.sync_copy(r0, acc.at[idx_v.at[c]], add=True)

            @pl.when(c + 2 < n_chunks)
            def _():
                start(c + 2, r0, s0)

            @pl.when(c + 1 < n_chunks)
            def _():
                wait(r1, s1)
                pltpu.sync_copy(r1, acc.at[idx_v.at[c + 1]], add=True)

        plsc.subcore_barrier()

        @pl.loop(sid, n_zc, step=_NS)
        def _(zc):
            pltpu.sync_copy(
                acc.at[pl.ds(zc * ZR, ZR)],
                out_hbm.at[pl.ds(zc * ZR, ZR), pl.ds(col0, Dh)])

    return k(vals, idx3)


def _edge_mlp(edges, gathered_i32, w1e, w1n_lo, w1n_hi, b1, w2, b2):
    E, DE = edges.shape
    G = gathered_i32.shape[1]
    H = w1e.shape[1]
    DO = w2.shape[1]
    BE = 8000
    nt = E // BE

    def body(e_ref, sn_ref, rn_ref, w1e_ref, w1lo_ref, w1hi_ref, b1_ref,
             w2_ref, b2_ref, out_ref):
        su = sn_ref[...]
        ru = rn_ref[...]
        mask = jnp.int32(-65536)
        s_lo = (lax.bitcast_convert_type(su << 16, jnp.float32)
                + lax.bitcast_convert_type(ru << 16, jnp.float32))
        s_hi = (lax.bitcast_convert_type(su & mask, jnp.float32)
                + lax.bitcast_convert_type(ru & mask, jnp.float32))
        x = jnp.dot(s_lo.astype(jnp.bfloat16), w1lo_ref[...],
                    preferred_element_type=jnp.float32)
        x += jnp.dot(s_hi.astype(jnp.bfloat16), w1hi_ref[...],
                     preferred_element_type=jnp.float32)
        x += jnp.dot(e_ref[...].astype(jnp.bfloat16), w1e_ref[...],
                     preferred_element_type=jnp.float32)
        x += b1_ref[...]
        h = jnp.maximum(x, 0.0).astype(jnp.bfloat16)
        y = jnp.dot(h, w2_ref[...], preferred_element_type=jnp.float32)
        y += b2_ref[...]
        out_ref[...] = y

    return pl.pallas_call(
        body,
        grid=(nt,),
        in_specs=[
            pl.BlockSpec((BE, DE), lambda i: (i, 0)),
            pl.BlockSpec((BE, G), lambda i: (i, 0)),
            pl.BlockSpec((BE, G), lambda i, n=nt: (n + i, 0)),
            pl.BlockSpec(w1e.shape, lambda i: (0, 0)),
            pl.BlockSpec(w1n_lo.shape, lambda i: (0, 0)),
            pl.BlockSpec(w1n_hi.shape, lambda i: (0, 0)),
            pl.BlockSpec((1, H), lambda i: (0, 0)),
            pl.BlockSpec(w2.shape, lambda i: (0, 0)),
            pl.BlockSpec((1, DO), lambda i: (0, 0)),
        ],
        out_specs=pl.BlockSpec((BE, DO), lambda i: (i, 0)),
        out_shape=jax.ShapeDtypeStruct((E, DO), jnp.float32),
        compiler_params=pltpu.CompilerParams(
            dimension_semantics=("parallel",)),
    )(edges, gathered_i32, gathered_i32, w1e, w1n_lo, w1n_hi, b1, w2, b2)


def _node_and_global_mlp(nodes, agg, w1n, w1a, b1, w2, b2,
                         g, inv_e, inv_n, wg1g, wg1e, wg1n, bg1_, wg2,
                         bg2_):
    N, DN = nodes.shape
    H = w1n.shape[1]
    DO = w2.shape[1]
    DGO = wg2.shape[1]
    DA = agg.shape[1]
    BN = 1000
    nt = N // BN

    def body(n_ref, a_ref, w1n_ref, w1a_ref, b1_ref, w2_ref, b2_ref,
             g_ref, wg1g_ref, wg1e_ref, wg1n_ref, bg1_ref, wg2_ref,
             bg2_ref, out_ref, go_ref, nsum_ref, esum_ref):
        i = pl.program_id(0)
        a = a_ref[...]
        x = jnp.dot(n_ref[...].astype(jnp.bfloat16), w1n_ref[...],
                    preferred_element_type=jnp.float32)
        x += jnp.dot(a.astype(jnp.bfloat16), w1a_ref[...],
                     preferred_element_type=jnp.float32)
        x += b1_ref[...]
        h = jnp.maximum(x, 0.0).astype(jnp.bfloat16)
        y = jnp.dot(h, w2_ref[...], preferred_element_type=jnp.float32)
        y += b2_ref[...]
        out_ref[...] = y
        part = jnp.sum(y, axis=0, keepdims=True)
        epart = jnp.sum(a, axis=0, keepdims=True)

        @pl.when(i == 0)
        def _():
            nsum_ref[...] = part
            esum_ref[...] = epart

        @pl.when(i > 0)
        def _():
            nsum_ref[...] += part
            esum_ref[...] += epart

        @pl.when(i == nt - 1)
        def _():
            gx = jnp.dot(g_ref[...], wg1g_ref[...],
                         preferred_element_type=jnp.float32)
            gx += jnp.dot(esum_ref[...] * inv_e, wg1e_ref[...],
                          preferred_element_type=jnp.float32)
            gx += jnp.dot(nsum_ref[...] * inv_n, wg1n_ref[...],
                          preferred_element_type=jnp.float32)
            gx += bg1_ref[...]
            gh = jnp.maximum(gx, 0.0)
            go_ref[...] = jnp.dot(gh, wg2_ref[...],
                                  preferred_element_type=jnp.float32)
            go_ref[...] += bg2_ref[...]

    full = lambda a: pl.BlockSpec(a.shape, lambda i: (0,) * a.ndim)
    return pl.pallas_call(
        body,
        grid=(nt,),
        in_specs=[
            pl.BlockSpec((BN, DN), lambda i: (i, 0)),
            pl.BlockSpec((BN, DA), lambda i: (i, 0)),
            full(w1n), full(w1a), pl.BlockSpec((1, H), lambda i: (0, 0)),
            full(w2), pl.BlockSpec((1, DO), lambda i: (0, 0)),
            full(g), full(wg1g), full(wg1e), full(wg1n),
            pl.BlockSpec((1, H), lambda i: (0, 0)), full(wg2),
            pl.BlockSpec((1, DGO), lambda i: (0, 0)),
        ],
        out_specs=[
            pl.BlockSpec((BN, DO), lambda i: (i, 0)),
            pl.BlockSpec((1, DGO), lambda i: (0, 0)),
            pl.BlockSpec((1, DO), lambda i: (0, 0)),
            pl.BlockSpec((1, DA), lambda i: (0, 0)),
        ],
        out_shape=[
            jax.ShapeDtypeStruct((N, DO), jnp.float32),
            jax.ShapeDtypeStruct((1, DGO), jnp.float32),
            jax.ShapeDtypeStruct((1, DO), jnp.float32),
            jax.ShapeDtypeStruct((1, DA), jnp.float32),
        ],
        compiler_params=pltpu.CompilerParams(
            dimension_semantics=("arbitrary",)),
    )(nodes, agg, w1n, w1a, b1, w2, b2, g, wg1g, wg1e, wg1n,
      bg1_.reshape(1, H), wg2, bg2_.reshape(1, DGO))


def kernel(nodes, edges, senders, receivers, globals_, We1, be1, We2, be2,
           Wn1, bn1, Wn2, bn2, Wg1, bg1, Wg2, bg2):
    N, DN = nodes.shape
    E, DE = edges.shape
    DG = globals_.shape[1]
    H = We1.shape[1]
    DE_OUT = We2.shape[1]
    DN_OUT = Wn2.shape[1]

    bf = jnp.bfloat16
    senders = senders.astype(jnp.int32)
    receivers = receivers.astype(jnp.int32)

    nodes_bf = nodes.astype(bf)
    Dh = DN // 2
    table_i32 = lax.bitcast_convert_type(
        jnp.stack([nodes_bf[:, :Dh], nodes_bf[:, Dh:]], axis=-1), jnp.int32)
    idx_all = jnp.concatenate([senders, receivers])
    gathered_i32 = _sc_gather_rows(table_i32, idx_all)

    w1e = We1[:DE].astype(bf)
    w1n_lo = (0.5 * We1[DE:DE + Dh]).astype(bf)
    w1n_hi = (0.5 * We1[DE + Dh:DE + DN]).astype(bf)
    b1e = (be1 + globals_[0] @ We1[DE + DN:]).reshape(1, H)
    new_edges = _edge_mlp(edges, gathered_i32, w1e, w1n_lo, w1n_hi,
                          b1e, We2.astype(bf), be2.reshape(1, DE_OUT))

    agg = _sc_segment_sum(new_edges, receivers, N, 0, E)

    wn1n = Wn1[:DN].astype(bf)
    wn1a = Wn1[DN:DN + DE_OUT].astype(bf)
    b1n = (bn1 + globals_[0] @ Wn1[DN + DE_OUT:]).reshape(1, H)
    wg1g = Wg1[:DG]
    wg1e = Wg1[DG:DG + DE_OUT]
    wg1n = Wg1[DG + DE_OUT:]
    new_nodes, new_globals, _, _ = _node_and_global_mlp(
        nodes, agg, wn1n, wn1a, b1n, Wn2.astype(bf), bn2.reshape(1, DN_OUT),
        globals_, 1.0 / E, 1.0 / N, wg1g, wg1e, wg1n, bg1, Wg2, bg2)

    return new_nodes, new_edges, new_globals
